# Initial kernel scaffold; baseline (speedup 1.0000x reference)
import jax
import jax.numpy as jnp
from jax.experimental import pallas as pl


def kernel(atom_features, bond_features, pair_indices, W, b):
    raise NotImplementedError("write your pallas kernel here")


# speedup vs baseline: 1.4331x; 1.4331x over previous
import functools

import jax
import jax.numpy as jnp
from jax import lax
from jax.experimental import pallas as pl
from jax.experimental.pallas import tpu as pltpu
from jax.experimental.pallas import tpu_sc as plsc

ATOM_DIM = 32
BOND_DIM = 16
N_NODES = 50000
N_EDGES = 100000

NC, NS = 2, 16
NW = NC * NS
CHUNK = 128
EP = 102400
NP_PAD = 51200
EB = 1024
KC = 20
KF = KC * ATOM_DIM


def _gather_call(src, atom):
    per_w = EP // NW
    n_chunks = per_w // CHUNK
    mesh = plsc.VectorSubcoreMesh(core_axis_name="c", subcore_axis_name="s")

    @functools.partial(
        pl.kernel,
        out_type=jax.ShapeDtypeStruct((EP, ATOM_DIM), jnp.float32),
        mesh=mesh,
        scratch_types=[
            pltpu.VMEM((CHUNK,), jnp.int32),
            pltpu.VMEM((CHUNK, ATOM_DIM), jnp.float32),
            pltpu.SemaphoreType.DMA,
        ],
        compiler_params=pltpu.CompilerParams(use_tc_tiling_on_sc=False),
    )
    def gather_k(src_hbm, atom_hbm, nbr_hbm, idx_v, rows_v, sem):
        wid = lax.axis_index("s") * NC + lax.axis_index("c")
        base = wid * per_w

        def body(k, _):
            off = base + k * CHUNK
            pltpu.sync_copy(src_hbm.at[pl.ds(off, CHUNK)], idx_v)
            pltpu.async_copy(atom_hbm.at[idx_v], rows_v, sem).wait()
            pltpu.sync_copy(rows_v, nbr_hbm.at[pl.ds(off, CHUNK)])
            return 0

        lax.fori_loop(0, n_chunks, body, 0)

    return gather_k(src, atom)


def _dense_call(bondp, nbr, w2):
    def body(bondp_ref, nbr_ref, w2_ref, out_ref, f_ref):
        nbr_b = nbr_ref[...]
        for c in range(KC):
            col = bondp_ref[:, c].reshape(EB, 1)
            f_ref[:, c * ATOM_DIM:(c + 1) * ATOM_DIM] = nbr_b * col
        out_ref[...] = jnp.dot(f_ref[...], w2_ref[...],
                               preferred_element_type=jnp.float32)

    return pl.pallas_call(
        body,
        grid=(EP // EB,),
        in_specs=[
            pl.BlockSpec((EB, ATOM_DIM), lambda i: (i, 0)),
            pl.BlockSpec((EB, ATOM_DIM), lambda i: (i, 0)),
            pl.BlockSpec((KF, ATOM_DIM), lambda i: (0, 0)),
        ],
        out_specs=pl.BlockSpec((EB, ATOM_DIM), lambda i: (i, 0)),
        out_shape=jax.ShapeDtypeStruct((EP, ATOM_DIM), jnp.float32),
        scratch_shapes=[pltpu.VMEM((EB, KF), jnp.float32)],
    )(bondp, nbr, w2)


def _scatter_call(dst, transformed, zrows):
    per_t = EP // NS
    n_e_chunks = per_t // CHUNK
    rows_t = NP_PAD // NS
    n_r_chunks = rows_t // CHUNK
    mesh = plsc.VectorSubcoreMesh(core_axis_name="c", subcore_axis_name="s")

    @functools.partial(
        pl.kernel,
        out_type=jax.ShapeDtypeStruct((NP_PAD, ATOM_DIM), jnp.float32),
        mesh=mesh,
        scratch_types=[
            pltpu.VMEM_SHARED((NP_PAD, ATOM_DIM), jnp.float32),
            pltpu.VMEM((CHUNK,), jnp.int32),
            pltpu.VMEM((CHUNK, ATOM_DIM), jnp.float32),
        ],
        compiler_params=pltpu.CompilerParams(use_tc_tiling_on_sc=False),
    )
    def scatter_k(dst_hbm, t_hbm, z_hbm, out_hbm, acc, idx_v, rows_v):
        cid = lax.axis_index("c")
        sid = lax.axis_index("s")

        @pl.when(cid == 0)
        def _():
            pltpu.sync_copy(z_hbm, rows_v)
            rbase = sid * rows_t

            def zbody(k, _):
                pltpu.sync_copy(rows_v, acc.at[pl.ds(rbase + k * CHUNK, CHUNK)])
                return 0

            lax.fori_loop(0, n_r_chunks, zbody, 0)

        plsc.subcore_barrier()

        @pl.when(cid == 0)
        def _():
            ebase = sid * per_t

            def sbody(k, _):
                off = ebase + k * CHUNK
                pltpu.sync_copy(dst_hbm.at[pl.ds(off, CHUNK)], idx_v)
                pltpu.sync_copy(t_hbm.at[pl.ds(off, CHUNK)], rows_v)
                pltpu.sync_copy(rows_v, acc.at[idx_v], add=True)
                return 0

            lax.fori_loop(0, n_e_chunks, sbody, 0)

        plsc.subcore_barrier()

        @pl.when(cid == 0)
        def _():
            rbase = sid * rows_t

            def obody(k, _):
                r = rbase + k * CHUNK
                pltpu.sync_copy(acc.at[pl.ds(r, CHUNK)], rows_v)
                pltpu.sync_copy(rows_v, out_hbm.at[pl.ds(r, CHUNK)])
                return 0

            lax.fori_loop(0, n_r_chunks, obody, 0)

    return scatter_k(dst, transformed, zrows)


def kernel(atom_features, bond_features, pair_indices, W, b):
    E = bond_features.shape[0]

    src = jnp.zeros((EP,), jnp.int32).at[:E].set(pair_indices[:, 1])
    dst = jnp.zeros((EP,), jnp.int32).at[:E].set(pair_indices[:, 0])
    bondp = (jnp.zeros((EP, ATOM_DIM), jnp.float32)
             .at[:E, :BOND_DIM].set(bond_features)
             .at[:E, BOND_DIM].set(1.0))

    w_aug = jnp.concatenate([W, b[:, None]], axis=1)
    w3 = w_aug.reshape(ATOM_DIM, ATOM_DIM, BOND_DIM + 1).transpose(2, 1, 0)
    w2p = jnp.zeros((KF, ATOM_DIM), jnp.float32).at[:(BOND_DIM + 1) * ATOM_DIM].set(
        w3.reshape((BOND_DIM + 1) * ATOM_DIM, ATOM_DIM))

    zrows = jnp.zeros((CHUNK, ATOM_DIM), jnp.float32)

    nbr = _gather_call(src, atom_features)
    transformed = _dense_call(bondp, nbr, w2p)
    out_pad = _scatter_call(dst, transformed, zrows)
    return out_pad[:N_NODES]


# speedup vs baseline: 2.5287x; 1.7646x over previous
import functools

import jax
import jax.numpy as jnp
from jax import lax
from jax.experimental import pallas as pl
from jax.experimental.pallas import tpu as pltpu
from jax.experimental.pallas import tpu_sc as plsc

ATOM_DIM = 32
BOND_DIM = 16
N_NODES = 50000
N_EDGES = 100000

NC, NS = 2, 16
NW = NC * NS
CHUNK = 128
NSUB = 5
SUPER = NSUB * CHUNK
EP = 102400
EPR = EP // CHUNK
NP_PAD = 51200
EB = 1024
KC = 20
KF = KC * ATOM_DIM


def _gather_call(src2, atom):
    supers_per_w = EP // NW // SUPER
    rows_per_w = EP // NW // CHUNK
    mesh = plsc.VectorSubcoreMesh(core_axis_name="c", subcore_axis_name="s")

    @functools.partial(
        pl.kernel,
        out_type=jax.ShapeDtypeStruct((EP, ATOM_DIM), jnp.float32),
        mesh=mesh,
        scratch_types=[
            pltpu.VMEM((NSUB, CHUNK), jnp.int32),
            pltpu.VMEM((SUPER, ATOM_DIM), jnp.float32),
            pltpu.SemaphoreType.DMA,
        ],
        compiler_params=pltpu.CompilerParams(use_tc_tiling_on_sc=False),
    )
    def gather_k(src_hbm, atom_hbm, nbr_hbm, idx_v, rows_v, sem):
        wid = lax.axis_index("s") * NC + lax.axis_index("c")
        ebase = wid * (EP // NW)
        rbase = wid * rows_per_w

        def body(s, _):
            pltpu.sync_copy(src_hbm.at[pl.ds(rbase + s * NSUB, NSUB)], idx_v)
            descs = [
                pltpu.async_copy(
                    atom_hbm.at[idx_v.at[b]],
                    rows_v.at[pl.ds(b * CHUNK, CHUNK)],
                    sem,
                )
                for b in range(NSUB)
            ]
            for d in descs:
                d.wait()
            pltpu.sync_copy(rows_v, nbr_hbm.at[pl.ds(ebase + s * SUPER, SUPER)])
            return 0

        lax.fori_loop(0, supers_per_w, body, 0)

    return gather_k(src2, atom)


def _dense_call(bondp, nbr, w2t):
    def body(bondp_ref, nbr_ref, w2t_ref, out_ref, ft_ref):
        nbr_t = nbr_ref[...].T
        bond_t = bondp_ref[...].T
        for c in range(KC):
            ft_ref[c * ATOM_DIM:(c + 1) * ATOM_DIM, :] = (
                nbr_t * bond_t[c:c + 1, :])
        t_t = jnp.dot(w2t_ref[...], ft_ref[...],
                      preferred_element_type=jnp.float32)
        out_ref[...] = t_t.T

    return pl.pallas_call(
        body,
        grid=(EP // EB,),
        in_specs=[
            pl.BlockSpec((EB, ATOM_DIM), lambda i: (i, 0)),
            pl.BlockSpec((EB, ATOM_DIM), lambda i: (i, 0)),
            pl.BlockSpec((ATOM_DIM, KF), lambda i: (0, 0)),
        ],
        out_specs=pl.BlockSpec((EB, ATOM_DIM), lambda i: (i, 0)),
        out_shape=jax.ShapeDtypeStruct((EP, ATOM_DIM), jnp.float32),
        scratch_shapes=[pltpu.VMEM((KF, EB), jnp.float32)],
    )(bondp, nbr, w2t)


def _scatter_call(dst2, transformed, zrows):
    supers_per_t = EP // NS // SUPER
    rows_per_t = EP // NS // CHUNK
    zchunks_per_t = NP_PAD // NS // SUPER
    mesh = plsc.VectorSubcoreMesh(core_axis_name="c", subcore_axis_name="s")

    @functools.partial(
        pl.kernel,
        out_type=jax.ShapeDtypeStruct((NP_PAD, ATOM_DIM), jnp.float32),
        mesh=mesh,
        scratch_types=[
            pltpu.VMEM_SHARED((NP_PAD, ATOM_DIM), jnp.float32),
            pltpu.VMEM((NSUB, CHUNK), jnp.int32),
            pltpu.VMEM((SUPER, ATOM_DIM), jnp.float32),
            pltpu.SemaphoreType.DMA,
        ],
        compiler_params=pltpu.CompilerParams(use_tc_tiling_on_sc=False),
    )
    def scatter_k(dst_hbm, t_hbm, z_hbm, out_hbm, acc, idx_v, rows_v, sem):
        cid = lax.axis_index("c")
        sid = lax.axis_index("s")

        @pl.when(cid == 0)
        def _():
            pltpu.sync_copy(z_hbm, rows_v)
            zbase = sid * (NP_PAD // NS)

            def zbody(q, _):
                pltpu.sync_copy(
                    rows_v, acc.at[pl.ds(zbase + q * SUPER, SUPER)])
                return 0

            lax.fori_loop(0, zchunks_per_t, zbody, 0)

        plsc.subcore_barrier()

        @pl.when(cid == 0)
        def _():
            ebase = sid * (EP // NS)
            rbase = sid * rows_per_t

            def sbody(s, _):
                pltpu.sync_copy(dst_hbm.at[pl.ds(rbase + s * NSUB, NSUB)],
                                idx_v)
                pltpu.sync_copy(t_hbm.at[pl.ds(ebase + s * SUPER, SUPER)],
                                rows_v)
                descs = [
                    pltpu.async_copy(
                        rows_v.at[pl.ds(b * CHUNK, CHUNK)],
                        acc.at[idx_v.at[b]],
                        sem,
                        add=True,
                    )
                    for b in range(NSUB)
                ]
                for d in descs:
                    d.wait()
                return 0

            lax.fori_loop(0, supers_per_t, sbody, 0)

        plsc.subcore_barrier()

        @pl.when(cid == 0)
        def _():
            zbase = sid * (NP_PAD // NS)

            def obody(q, _):
                r = zbase + q * SUPER
                pltpu.sync_copy(acc.at[pl.ds(r, SUPER)], rows_v)
                pltpu.sync_copy(rows_v, out_hbm.at[pl.ds(r, SUPER)])
                return 0

            lax.fori_loop(0, zchunks_per_t, obody, 0)

    return scatter_k(dst2, transformed, zrows)


def kernel(atom_features, bond_features, pair_indices, W, b):
    E = bond_features.shape[0]

    pad = jnp.zeros((EP - E,), jnp.int32)
    src2 = jnp.concatenate([pair_indices[:, 1], pad]).reshape(EPR, CHUNK)
    dst2 = jnp.concatenate([pair_indices[:, 0], pad]).reshape(EPR, CHUNK)
    bondp = (jnp.zeros((EP, ATOM_DIM), jnp.float32)
             .at[:E, :BOND_DIM].set(bond_features)
             .at[:E, BOND_DIM].set(1.0))

    w_aug = jnp.concatenate([W, b[:, None]], axis=1)
    w3 = w_aug.reshape(ATOM_DIM, ATOM_DIM, BOND_DIM + 1).transpose(2, 1, 0)
    w2t = jnp.zeros((ATOM_DIM, KF), jnp.float32).at[:, :(BOND_DIM + 1) * ATOM_DIM].set(
        w3.reshape((BOND_DIM + 1) * ATOM_DIM, ATOM_DIM).T)

    zrows = jnp.zeros((SUPER, ATOM_DIM), jnp.float32)

    nbr = _gather_call(src2, atom_features)
    transformed = _dense_call(bondp, nbr, w2t)
    out_pad = _scatter_call(dst2, transformed, zrows)
    return out_pad[:N_NODES]


# speedup vs baseline: 2.6006x; 1.0284x over previous
import functools

import jax
import jax.numpy as jnp
from jax import lax
from jax.experimental import pallas as pl
from jax.experimental.pallas import tpu as pltpu
from jax.experimental.pallas import tpu_sc as plsc

ATOM_DIM = 32
BOND_DIM = 16
N_NODES = 50000
N_EDGES = 100000

NC, NS = 2, 16
NW = NC * NS
CHUNK = 128
NSUB = 5
SUPER = NSUB * CHUNK
EP = 102400
EPR = EP // CHUNK
NP_PAD = 51200
EB = 1024
KC = 20
KF = KC * ATOM_DIM


def _gather_call(src2, atom):
    rows_per_w = EP // NW // CHUNK
    per_w = EP // NW
    mesh = plsc.VectorSubcoreMesh(core_axis_name="c", subcore_axis_name="s")

    @functools.partial(
        pl.kernel,
        out_type=jax.ShapeDtypeStruct((EP, ATOM_DIM), jnp.float32),
        mesh=mesh,
        scratch_types=[
            pltpu.VMEM((rows_per_w, CHUNK), jnp.int32),
            pltpu.VMEM((per_w, ATOM_DIM), jnp.float32),
            pltpu.SemaphoreType.DMA,
        ],
        compiler_params=pltpu.CompilerParams(use_tc_tiling_on_sc=False),
    )
    def gather_k(src_hbm, atom_hbm, nbr_hbm, idx_v, rows_v, sem):
        wid = lax.axis_index("s") * NC + lax.axis_index("c")
        pltpu.sync_copy(src_hbm.at[pl.ds(wid * rows_per_w, rows_per_w)], idx_v)
        descs = [
            pltpu.async_copy(
                atom_hbm.at[idx_v.at[r]],
                rows_v.at[pl.ds(r * CHUNK, CHUNK)],
                sem,
            )
            for r in range(rows_per_w)
        ]
        for d in descs:
            d.wait()
        pltpu.sync_copy(rows_v, nbr_hbm.at[pl.ds(wid * per_w, per_w)])

    return gather_k(src2, atom)


def _dense_call(bondp, nbr, w2t):
    def body(bondp_ref, nbr_ref, w2t_ref, out_ref, ft_ref):
        nbr_t = nbr_ref[...].T
        bond_t = bondp_ref[...].T
        for c in range(KC):
            ft_ref[c * ATOM_DIM:(c + 1) * ATOM_DIM, :] = (
                nbr_t * bond_t[c:c + 1, :])
        t_t = jnp.dot(w2t_ref[...], ft_ref[...],
                      preferred_element_type=jnp.float32)
        out_ref[...] = t_t.T

    return pl.pallas_call(
        body,
        grid=(EP // EB,),
        in_specs=[
            pl.BlockSpec((EB, ATOM_DIM), lambda i: (i, 0)),
            pl.BlockSpec((EB, ATOM_DIM), lambda i: (i, 0)),
            pl.BlockSpec((ATOM_DIM, KF), lambda i: (0, 0)),
        ],
        out_specs=pl.BlockSpec((EB, ATOM_DIM), lambda i: (i, 0)),
        out_shape=jax.ShapeDtypeStruct((EP, ATOM_DIM), jnp.float32),
        scratch_shapes=[pltpu.VMEM((KF, EB), jnp.float32)],
    )(bondp, nbr, w2t)


def _scatter_call(dst2, transformed, zrows):
    supers_per_t = EP // NS // SUPER
    acc_per_t = NP_PAD // NS
    nz = acc_per_t // SUPER
    out_per_t = N_NODES // NS
    oc = out_per_t // 5
    mesh = plsc.VectorSubcoreMesh(core_axis_name="c", subcore_axis_name="s")

    @functools.partial(
        pl.kernel,
        out_type=jax.ShapeDtypeStruct((N_NODES, ATOM_DIM), jnp.float32),
        mesh=mesh,
        scratch_types=[
            pltpu.VMEM_SHARED((NP_PAD, ATOM_DIM), jnp.float32),
            pltpu.VMEM((NSUB, CHUNK), jnp.int32),
            pltpu.VMEM((SUPER, ATOM_DIM), jnp.float32),
            pltpu.SemaphoreType.DMA,
        ],
        compiler_params=pltpu.CompilerParams(use_tc_tiling_on_sc=False),
    )
    def scatter_k(dst_hbm, t_hbm, z_hbm, out_hbm, acc, idx_v, rows_v, sem):
        cid = lax.axis_index("c")
        sid = lax.axis_index("s")

        @pl.when(cid == 0)
        def _():
            pltpu.sync_copy(z_hbm, rows_v)
            zbase = sid * acc_per_t
            for q in range(nz):
                pltpu.sync_copy(rows_v,
                                acc.at[pl.ds(zbase + q * SUPER, SUPER)])

        plsc.subcore_barrier()

        @pl.when(cid == 0)
        def _():
            rbase = sid * (supers_per_t * NSUB)

            def sbody(s, _):
                pltpu.sync_copy(
                    dst_hbm.at[pl.ds(rbase + s * NSUB, NSUB)], idx_v)
                ebase = (rbase + s * NSUB) * CHUNK
                pltpu.sync_copy(t_hbm.at[pl.ds(ebase, SUPER)], rows_v)
                descs = [
                    pltpu.async_copy(
                        rows_v.at[pl.ds(r * CHUNK, CHUNK)],
                        acc.at[idx_v.at[r]],
                        sem,
                        add=True,
                    )
                    for r in range(NSUB)
                ]
                for d in descs:
                    d.wait()
                return 0

            lax.fori_loop(0, supers_per_t, sbody, 0)

        plsc.subcore_barrier()

        @pl.when(cid == 0)
        def _():
            obase = sid * out_per_t
            for q in range(5):
                r = obase + q * oc
                pltpu.sync_copy(acc.at[pl.ds(r, oc)],
                                rows_v.at[pl.ds(0, oc)])
                pltpu.sync_copy(rows_v.at[pl.ds(0, oc)],
                                out_hbm.at[pl.ds(r, oc)])

    return scatter_k(dst2, transformed, zrows)


def kernel(atom_features, bond_features, pair_indices, W, b):
    E = bond_features.shape[0]

    pad = jnp.zeros((EP - E,), jnp.int32)
    src2 = jnp.concatenate([pair_indices[:, 1], pad]).reshape(EPR, CHUNK)
    dst2 = jnp.concatenate([pair_indices[:, 0], pad]).reshape(EPR, CHUNK)
    bondp = (jnp.zeros((EP, ATOM_DIM), jnp.float32)
             .at[:E, :BOND_DIM].set(bond_features)
             .at[:E, BOND_DIM].set(1.0))

    w_aug = jnp.concatenate([W, b[:, None]], axis=1)
    w3 = w_aug.reshape(ATOM_DIM, ATOM_DIM, BOND_DIM + 1).transpose(2, 1, 0)
    w2t = jnp.zeros((ATOM_DIM, KF), jnp.float32).at[:, :(BOND_DIM + 1) * ATOM_DIM].set(
        w3.reshape((BOND_DIM + 1) * ATOM_DIM, ATOM_DIM).T)

    zrows = jnp.zeros((SUPER, ATOM_DIM), jnp.float32)

    nbr = _gather_call(src2, atom_features)
    transformed = _dense_call(bondp, nbr, w2t)
    return _scatter_call(dst2, transformed, zrows)


# speedup vs baseline: 3.2187x; 1.2377x over previous
import functools

import jax
import jax.numpy as jnp
from jax import lax
from jax.experimental import pallas as pl
from jax.experimental.pallas import tpu as pltpu
from jax.experimental.pallas import tpu_sc as plsc

ATOM_DIM = 32
BOND_DIM = 16
N_NODES = 50000
N_EDGES = 100000

NC, NS = 2, 16
NW = NC * NS
CHUNK = 128
EP = 102400
PAIR_PAD = 2 * EP
EB = 1024
KF = (BOND_DIM + 1) * ATOM_DIM

NHALF = 25600
ACC_ROWS = NHALF + 8


def _gather_call(pair_pad, atom):
    per_w = EP // NW
    n_streams = per_w // CHUNK
    mesh = plsc.VectorSubcoreMesh(core_axis_name="c", subcore_axis_name="s")

    @functools.partial(
        pl.kernel,
        out_type=jax.ShapeDtypeStruct((EP, ATOM_DIM), jnp.float32),
        mesh=mesh,
        scratch_types=[
            pltpu.VMEM((2 * per_w,), jnp.int32),
            pltpu.VMEM((per_w,), jnp.int32),
            pltpu.VMEM((per_w, ATOM_DIM), jnp.float32),
            pltpu.SemaphoreType.DMA,
        ],
        compiler_params=pltpu.CompilerParams(use_tc_tiling_on_sc=False, needs_layout_passes=False),
    )
    def gather_k(pair_hbm, atom_hbm, nbr_hbm, pair_v, idx_v, rows_v, sem):
        wid = lax.axis_index("s") * NC + lax.axis_index("c")
        ebase = wid * per_w
        pltpu.sync_copy(pair_hbm.at[pl.ds(ebase * 2, 2 * per_w)], pair_v)
        lanes = lax.iota(jnp.int32, 16)

        def cbody(i, _):
            pos = 2 * (i * 16 + lanes) + 1
            v = plsc.load_gather(pair_v, [pos])
            idx_v[pl.ds(i * 16, 16)] = v
            return 0

        lax.fori_loop(0, per_w // 16, cbody, 0)

        descs = [
            pltpu.async_copy(
                atom_hbm.at[idx_v.at[pl.ds(r * CHUNK, CHUNK)]],
                rows_v.at[pl.ds(r * CHUNK, CHUNK)],
                sem,
            )
            for r in range(n_streams)
        ]
        for d in descs:
            d.wait()
        pltpu.sync_copy(rows_v, nbr_hbm.at[pl.ds(ebase, per_w)])

    return gather_k(pair_pad, atom)


def _dense_call(bond, nbr, w2t):
    last_block = (bond.shape[0] - 1) // EB

    def body(bond_ref, nbr_ref, w2t_ref, out_ref, ft_ref):
        nbr_t = nbr_ref[...].T
        bond_t = bond_ref[...].T
        for c in range(BOND_DIM):
            ft_ref[c * ATOM_DIM:(c + 1) * ATOM_DIM, :] = (
                nbr_t * bond_t[c:c + 1, :])
        ft_ref[BOND_DIM * ATOM_DIM:, :] = nbr_t
        t_t = jnp.dot(w2t_ref[...], ft_ref[...],
                      preferred_element_type=jnp.float32)
        out_ref[...] = t_t.T

    return pl.pallas_call(
        body,
        grid=(EP // EB,),
        in_specs=[
            pl.BlockSpec((EB, BOND_DIM),
                         lambda i: (jnp.minimum(i, last_block), 0)),
            pl.BlockSpec((EB, ATOM_DIM), lambda i: (i, 0)),
            pl.BlockSpec((ATOM_DIM, KF), lambda i: (0, 0)),
        ],
        out_specs=pl.BlockSpec((EB, ATOM_DIM), lambda i: (i, 0)),
        out_shape=jax.ShapeDtypeStruct((EP, ATOM_DIM), jnp.float32),
        scratch_shapes=[pltpu.VMEM((KF, EB), jnp.float32)],
    )(bond, nbr, w2t)


def _scatter_call(pair_pad, transformed):
    per_t = EP // NS
    n_slabs = 5
    slab = per_t // n_slabs
    idx_rows = per_t // CHUNK
    zc0 = NHALF // NS
    zc1_total = N_NODES - NHALF
    zc1 = zc1_total // NS
    mesh = plsc.VectorSubcoreMesh(core_axis_name="c", subcore_axis_name="s")

    @functools.partial(
        pl.kernel,
        out_type=jax.ShapeDtypeStruct((N_NODES, ATOM_DIM), jnp.float32),
        mesh=mesh,
        scratch_types=[
            pltpu.VMEM_SHARED((ACC_ROWS, ATOM_DIM), jnp.float32),
            pltpu.VMEM((2 * per_t,), jnp.int32),
            pltpu.VMEM((idx_rows, CHUNK), jnp.int32),
            pltpu.VMEM((slab, ATOM_DIM), jnp.float32),
            pltpu.SemaphoreType.DMA,
        ],
        compiler_params=pltpu.CompilerParams(use_tc_tiling_on_sc=False, needs_layout_passes=False),
    )
    def scatter_k(pair_hbm, t_hbm, out_hbm, acc, pair_v, idx_v, rows_v, sem):
        cid = lax.axis_index("c")
        sid = lax.axis_index("s")
        lanes = lax.iota(jnp.int32, 16)

        def zvec(i, _):
            rows_v[i // 2, pl.ds((i % 2) * 16, 16)] = jnp.zeros(
                (16,), jnp.float32)
            return 0

        lax.fori_loop(0, 1280, zvec, 0)
        zbase = sid * zc0
        for (off, n) in ((0, 640), (640, 640), (1280, 320)):
            pltpu.sync_copy(rows_v.at[pl.ds(0, n)],
                            acc.at[pl.ds(zbase + off, n)])

        plsc.subcore_barrier()

        pltpu.sync_copy(pair_hbm.at[pl.ds(sid * per_t * 2, 2 * per_t)],
                        pair_v)
        ebase = sid * per_t
        nlo = cid * NHALF

        def cbody(j, _):
            e = j * 16 + lanes
            d = plsc.load_gather(pair_v, [2 * e])
            local = d - nlo
            valid = ((local >= 0) & (local < NHALF)
                     & (ebase + e < N_EDGES))
            lv = jnp.where(valid, local, NHALF)
            idx_v[j // 8, pl.ds((j % 8) * 16, 16)] = lv
            return 0

        lax.fori_loop(0, per_t // 16, cbody, 0)

        for sl in range(n_slabs):
            pltpu.sync_copy(t_hbm.at[pl.ds(ebase + sl * slab, slab)], rows_v)
            descs = [
                pltpu.async_copy(
                    rows_v.at[pl.ds(r * CHUNK, CHUNK)],
                    acc.at[idx_v.at[sl * (slab // CHUNK) + r]],
                    sem,
                    add=True,
                )
                for r in range(slab // CHUNK)
            ]
            for d in descs:
                d.wait()

        plsc.subcore_barrier()

        @pl.when(cid == 0)
        def _():
            for (off, n) in ((0, 640), (640, 640), (1280, 320)):
                r = sid * zc0 + off
                pltpu.sync_copy(acc.at[pl.ds(r, n)], rows_v.at[pl.ds(0, n)])
                pltpu.sync_copy(rows_v.at[pl.ds(0, n)],
                                out_hbm.at[pl.ds(r, n)])

        @pl.when(cid == 1)
        def _():
            for (off, n) in ((0, 640), (640, 640), (1280, 245)):
                r = sid * zc1 + off
                pltpu.sync_copy(acc.at[pl.ds(r, n)], rows_v.at[pl.ds(0, n)])
                pltpu.sync_copy(rows_v.at[pl.ds(0, n)],
                                out_hbm.at[pl.ds(NHALF + r, n)])

    return scatter_k(pair_pad, transformed)


def kernel(atom_features, bond_features, pair_indices, W, b):
    pair_flat = pair_indices.reshape(-1)
    pair_pad = jnp.concatenate(
        [pair_flat, jnp.zeros((PAIR_PAD - pair_flat.shape[0],), jnp.int32)])

    w_aug = jnp.concatenate([W, b[:, None]], axis=1)
    w3 = w_aug.reshape(ATOM_DIM, ATOM_DIM, BOND_DIM + 1).transpose(2, 1, 0)
    w2t = w3.reshape(KF, ATOM_DIM).T

    nbr = _gather_call(pair_pad, atom_features)
    transformed = _dense_call(bond_features, nbr, w2t)
    return _scatter_call(pair_pad, transformed)


# speedup vs baseline: 3.9494x; 1.2270x over previous
import functools

import jax
import jax.numpy as jnp
from jax import lax
from jax.experimental import pallas as pl
from jax.experimental.pallas import tpu as pltpu
from jax.experimental.pallas import tpu_sc as plsc

ATOM_DIM = 32
BOND_DIM = 16
N_NODES = 50000
N_EDGES = 100000

NC, NS = 2, 16
NW = NC * NS
CHUNK = 128
EP = 102400
PAIR_PAD = 2 * EP
EB = 1024
KF = (BOND_DIM + 1) * ATOM_DIM

NHALF = 25600
ACC_ROWS = NHALF + 8


def _gather_call(src_pad, atom):
    per_w = EP // NW
    n_streams = per_w // CHUNK
    mesh = plsc.VectorSubcoreMesh(core_axis_name="c", subcore_axis_name="s")

    @functools.partial(
        pl.kernel,
        out_type=jax.ShapeDtypeStruct((EP, ATOM_DIM), jnp.float32),
        mesh=mesh,
        scratch_types=[
            pltpu.VMEM((per_w,), jnp.int32),
            pltpu.VMEM((per_w, ATOM_DIM), jnp.float32),
            pltpu.SemaphoreType.DMA,
        ],
        compiler_params=pltpu.CompilerParams(use_tc_tiling_on_sc=False, needs_layout_passes=False),
    )
    def gather_k(src_hbm, atom_hbm, nbr_hbm, idx_v, rows_v, sem):
        wid = lax.axis_index("s") * NC + lax.axis_index("c")
        ebase = wid * per_w
        pltpu.sync_copy(src_hbm.at[pl.ds(ebase, per_w)], idx_v)

        descs = [
            pltpu.async_copy(
                atom_hbm.at[idx_v.at[pl.ds(r * CHUNK, CHUNK)]],
                rows_v.at[pl.ds(r * CHUNK, CHUNK)],
                sem,
            )
            for r in range(n_streams)
        ]
        for d in descs:
            d.wait()
        pltpu.sync_copy(rows_v, nbr_hbm.at[pl.ds(ebase, per_w)])

    return gather_k(src_pad, atom)


def _dense_call(bond_t_all, nbr, w2t):
    last_block = (bond_t_all.shape[1] - 1) // EB

    def body(bond_ref, nbr_ref, w2t_ref, out_ref, ft_ref):
        nbr_t = nbr_ref[...].T
        bond_t = bond_ref[...]
        for c in range(BOND_DIM):
            ft_ref[c * ATOM_DIM:(c + 1) * ATOM_DIM, :] = (
                nbr_t * bond_t[c:c + 1, :])
        ft_ref[BOND_DIM * ATOM_DIM:, :] = nbr_t
        t_t = jnp.dot(w2t_ref[...], ft_ref[...],
                      preferred_element_type=jnp.float32)
        out_ref[...] = t_t.T

    return pl.pallas_call(
        body,
        grid=(EP // EB,),
        in_specs=[
            pl.BlockSpec((BOND_DIM, EB),
                         lambda i: (0, jnp.minimum(i, last_block))),
            pl.BlockSpec((EB, ATOM_DIM), lambda i: (i, 0)),
            pl.BlockSpec((ATOM_DIM, KF), lambda i: (0, 0)),
        ],
        out_specs=pl.BlockSpec((EB, ATOM_DIM), lambda i: (i, 0)),
        out_shape=jax.ShapeDtypeStruct((EP, ATOM_DIM), jnp.float32),
        scratch_shapes=[pltpu.VMEM((KF, EB), jnp.float32)],
    )(bond_t_all, nbr, w2t)


def _scatter_call(dst_pad, transformed):
    per_t = EP // NS
    n_slabs = 5
    slab = per_t // n_slabs
    idx_rows = per_t // CHUNK
    zc0 = NHALF // NS
    zc1_total = N_NODES - NHALF
    zc1 = zc1_total // NS
    mesh = plsc.VectorSubcoreMesh(core_axis_name="c", subcore_axis_name="s")

    @functools.partial(
        pl.kernel,
        out_type=jax.ShapeDtypeStruct((N_NODES, ATOM_DIM), jnp.float32),
        mesh=mesh,
        scratch_types=[
            pltpu.VMEM_SHARED((ACC_ROWS, ATOM_DIM), jnp.float32),
            pltpu.VMEM((per_t,), jnp.int32),
            pltpu.VMEM((idx_rows, CHUNK), jnp.int32),
            pltpu.VMEM((slab, ATOM_DIM), jnp.float32),
            pltpu.SemaphoreType.DMA,
        ],
        compiler_params=pltpu.CompilerParams(use_tc_tiling_on_sc=False, needs_layout_passes=False),
    )
    def scatter_k(dst_hbm, t_hbm, out_hbm, acc, dst_v, idx_v, rows_v, sem):
        cid = lax.axis_index("c")
        sid = lax.axis_index("s")
        lanes = lax.iota(jnp.int32, 16)

        def zvec(i, _):
            rows_v[i // 2, pl.ds((i % 2) * 16, 16)] = jnp.zeros(
                (16,), jnp.float32)
            return 0

        lax.fori_loop(0, 1280, zvec, 0)
        zbase = sid * zc0
        for (off, n) in ((0, 640), (640, 640), (1280, 320)):
            pltpu.sync_copy(rows_v.at[pl.ds(0, n)],
                            acc.at[pl.ds(zbase + off, n)])

        plsc.subcore_barrier()

        pltpu.sync_copy(dst_hbm.at[pl.ds(sid * per_t, per_t)], dst_v)
        ebase = sid * per_t
        nlo = cid * NHALF

        def cbody(j, _):
            e = j * 16 + lanes
            d = dst_v[pl.ds(j * 16, 16)]
            local = d - nlo
            valid = ((local >= 0) & (local < NHALF)
                     & (ebase + e < N_EDGES))
            lv = jnp.where(valid, local, NHALF)
            idx_v[j // 8, pl.ds((j % 8) * 16, 16)] = lv
            return 0

        lax.fori_loop(0, per_t // 16, cbody, 0)

        for sl in range(n_slabs):
            pltpu.sync_copy(t_hbm.at[pl.ds(ebase + sl * slab, slab)], rows_v)
            descs = [
                pltpu.async_copy(
                    rows_v.at[pl.ds(r * CHUNK, CHUNK)],
                    acc.at[idx_v.at[sl * (slab // CHUNK) + r]],
                    sem,
                    add=True,
                )
                for r in range(slab // CHUNK)
            ]
            for d in descs:
                d.wait()

        plsc.subcore_barrier()

        @pl.when(cid == 0)
        def _():
            for (off, n) in ((0, 640), (640, 640), (1280, 320)):
                r = sid * zc0 + off
                pltpu.sync_copy(acc.at[pl.ds(r, n)], rows_v.at[pl.ds(0, n)])
                pltpu.sync_copy(rows_v.at[pl.ds(0, n)],
                                out_hbm.at[pl.ds(r, n)])

        @pl.when(cid == 1)
        def _():
            for (off, n) in ((0, 640), (640, 640), (1280, 245)):
                r = sid * zc1 + off
                pltpu.sync_copy(acc.at[pl.ds(r, n)], rows_v.at[pl.ds(0, n)])
                pltpu.sync_copy(rows_v.at[pl.ds(0, n)],
                                out_hbm.at[pl.ds(NHALF + r, n)])

    return scatter_k(dst_pad, transformed)


def kernel(atom_features, bond_features, pair_indices, W, b):
    E = pair_indices.shape[0]
    zpad = jnp.zeros((EP - E,), jnp.int32)
    src_pad = jnp.concatenate([pair_indices[:, 1], zpad])
    dst_pad = jnp.concatenate([pair_indices[:, 0], zpad])

    w_aug = jnp.concatenate([W, b[:, None]], axis=1)
    w3 = w_aug.reshape(ATOM_DIM, ATOM_DIM, BOND_DIM + 1).transpose(2, 1, 0)
    w2t = w3.reshape(KF, ATOM_DIM).T

    nbr = _gather_call(src_pad, atom_features)
    transformed = _dense_call(bond_features.T, nbr, w2t)
    return _scatter_call(dst_pad, transformed)


# speedup vs baseline: 4.4406x; 1.1244x over previous
import functools

import jax
import jax.numpy as jnp
from jax import lax
from jax.experimental import pallas as pl
from jax.experimental.pallas import tpu as pltpu
from jax.experimental.pallas import tpu_sc as plsc

ATOM_DIM = 32
BOND_DIM = 16
N_NODES = 50000
N_EDGES = 100000

NC, NS = 2, 16
NW = NC * NS
CHUNK = 128
EP = 102400
PAIR_PAD = 2 * EP
EB = 1024
KF = (BOND_DIM + 1) * ATOM_DIM

NHALF = 25600
NDUMMY = 128
ACC_ROWS = NHALF + NDUMMY


def _gather_call(src_pad, atom):
    per_w = EP // NW
    n_streams = per_w // CHUNK
    mesh = plsc.VectorSubcoreMesh(core_axis_name="c", subcore_axis_name="s")

    @functools.partial(
        pl.kernel,
        out_type=jax.ShapeDtypeStruct((EP, ATOM_DIM), jnp.float32),
        mesh=mesh,
        scratch_types=[
            pltpu.VMEM((per_w,), jnp.int32),
            pltpu.VMEM((per_w, ATOM_DIM), jnp.float32),
            pltpu.SemaphoreType.DMA,
        ],
        compiler_params=pltpu.CompilerParams(use_tc_tiling_on_sc=False, needs_layout_passes=False),
    )
    def gather_k(src_hbm, atom_hbm, nbr_hbm, idx_v, rows_v, sem):
        wid = lax.axis_index("s") * NC + lax.axis_index("c")
        ebase = wid * per_w
        pltpu.sync_copy(src_hbm.at[pl.ds(ebase, per_w)], idx_v)

        descs = [
            pltpu.async_copy(
                atom_hbm.at[idx_v.at[pl.ds(r * CHUNK, CHUNK)]],
                rows_v.at[pl.ds(r * CHUNK, CHUNK)],
                sem,
            )
            for r in range(n_streams)
        ]
        for d in descs:
            d.wait()
        pltpu.sync_copy(rows_v, nbr_hbm.at[pl.ds(ebase, per_w)])

    return gather_k(src_pad, atom)


def _dense_call(bond_t_all, nbr, w2t):
    last_block = (bond_t_all.shape[1] - 1) // EB

    def body(bond_ref, nbr_ref, w2t_ref, out_ref, ft_ref):
        nbr_t = nbr_ref[...].T
        bond_t = bond_ref[...]
        for c in range(BOND_DIM):
            ft_ref[c * ATOM_DIM:(c + 1) * ATOM_DIM, :] = (
                nbr_t * bond_t[c:c + 1, :]).astype(jnp.bfloat16)
        ft_ref[BOND_DIM * ATOM_DIM:, :] = nbr_t.astype(jnp.bfloat16)
        t_t = jnp.dot(w2t_ref[...], ft_ref[...],
                      preferred_element_type=jnp.float32)
        out_ref[...] = t_t.T

    return pl.pallas_call(
        body,
        grid=(EP // EB,),
        in_specs=[
            pl.BlockSpec((BOND_DIM, EB),
                         lambda i: (0, jnp.minimum(i, last_block))),
            pl.BlockSpec((EB, ATOM_DIM), lambda i: (i, 0)),
            pl.BlockSpec((ATOM_DIM, KF), lambda i: (0, 0)),
        ],
        out_specs=pl.BlockSpec((EB, ATOM_DIM), lambda i: (i, 0)),
        out_shape=jax.ShapeDtypeStruct((EP, ATOM_DIM), jnp.float32),
        scratch_shapes=[pltpu.VMEM((KF, EB), jnp.bfloat16)],
    )(bond_t_all, nbr, w2t)


def _scatter_call(dst_pad, transformed):
    per_t = EP // NS
    n_slabs = 5
    slab = per_t // n_slabs
    idx_rows = per_t // CHUNK
    zc0 = NHALF // NS
    zc1_total = N_NODES - NHALF
    zc1 = zc1_total // NS
    mesh = plsc.VectorSubcoreMesh(core_axis_name="c", subcore_axis_name="s")

    @functools.partial(
        pl.kernel,
        out_type=jax.ShapeDtypeStruct((N_NODES, ATOM_DIM), jnp.float32),
        mesh=mesh,
        scratch_types=[
            pltpu.VMEM_SHARED((ACC_ROWS, ATOM_DIM), jnp.float32),
            pltpu.VMEM((per_t,), jnp.int32),
            pltpu.VMEM((idx_rows, CHUNK), jnp.int32),
            pltpu.VMEM((slab, ATOM_DIM), jnp.float32),
            pltpu.SemaphoreType.DMA,
        ],
        compiler_params=pltpu.CompilerParams(use_tc_tiling_on_sc=False, needs_layout_passes=False),
    )
    def scatter_k(dst_hbm, t_hbm, out_hbm, acc, dst_v, idx_v, rows_v, sem):
        cid = lax.axis_index("c")
        sid = lax.axis_index("s")
        lanes = lax.iota(jnp.int32, 16)

        def zvec(i, _):
            rows_v[i // 2, pl.ds((i % 2) * 16, 16)] = jnp.zeros(
                (16,), jnp.float32)
            return 0

        lax.fori_loop(0, 1280, zvec, 0)
        zbase = sid * zc0
        for (off, n) in ((0, 640), (640, 640), (1280, 320)):
            pltpu.sync_copy(rows_v.at[pl.ds(0, n)],
                            acc.at[pl.ds(zbase + off, n)])

        plsc.subcore_barrier()

        pltpu.sync_copy(dst_hbm.at[pl.ds(sid * per_t, per_t)], dst_v)
        ebase = sid * per_t
        nlo = cid * NHALF

        def cbody(j, _):
            e = j * 16 + lanes
            d = dst_v[pl.ds(j * 16, 16)]
            local = d - nlo
            valid = ((local >= 0) & (local < NHALF)
                     & (ebase + e < N_EDGES))
            lv = jnp.where(valid, local, NHALF + (e & (NDUMMY - 1)))
            idx_v[j // 8, pl.ds((j % 8) * 16, 16)] = lv
            return 0

        lax.fori_loop(0, per_t // 16, cbody, 0)

        for sl in range(n_slabs):
            pltpu.sync_copy(t_hbm.at[pl.ds(ebase + sl * slab, slab)], rows_v)
            descs = [
                pltpu.async_copy(
                    rows_v.at[pl.ds(r * CHUNK, CHUNK)],
                    acc.at[idx_v.at[sl * (slab // CHUNK) + r]],
                    sem,
                    add=True,
                )
                for r in range(slab // CHUNK)
            ]
            for d in descs:
                d.wait()

        plsc.subcore_barrier()

        @pl.when(cid == 0)
        def _():
            for (off, n) in ((0, 640), (640, 640), (1280, 320)):
                r = sid * zc0 + off
                pltpu.sync_copy(acc.at[pl.ds(r, n)], rows_v.at[pl.ds(0, n)])
                pltpu.sync_copy(rows_v.at[pl.ds(0, n)],
                                out_hbm.at[pl.ds(r, n)])

        @pl.when(cid == 1)
        def _():
            for (off, n) in ((0, 640), (640, 640), (1280, 245)):
                r = sid * zc1 + off
                pltpu.sync_copy(acc.at[pl.ds(r, n)], rows_v.at[pl.ds(0, n)])
                pltpu.sync_copy(rows_v.at[pl.ds(0, n)],
                                out_hbm.at[pl.ds(NHALF + r, n)])

    return scatter_k(dst_pad, transformed)


def kernel(atom_features, bond_features, pair_indices, W, b):
    E = pair_indices.shape[0]
    zpad = jnp.zeros((EP - E,), jnp.int32)
    src_pad = jnp.concatenate([pair_indices[:, 1], zpad])
    dst_pad = jnp.concatenate([pair_indices[:, 0], zpad])

    w_aug = jnp.concatenate([W, b[:, None]], axis=1)
    w3 = w_aug.reshape(ATOM_DIM, ATOM_DIM, BOND_DIM + 1).transpose(2, 1, 0)
    w2t = w3.reshape(KF, ATOM_DIM).T.astype(jnp.bfloat16)

    nbr = _gather_call(src_pad, atom_features)
    transformed = _dense_call(bond_features.T, nbr, w2t)
    return _scatter_call(dst_pad, transformed)


# speedup vs baseline: 4.9654x; 1.1182x over previous
import functools

import jax
import jax.numpy as jnp
from jax import lax
from jax.experimental import pallas as pl
from jax.experimental.pallas import tpu as pltpu
from jax.experimental.pallas import tpu_sc as plsc

ATOM_DIM = 32
BOND_DIM = 16
N_NODES = 50000
N_EDGES = 100000

NC, NS = 2, 16
NW = NC * NS
CHUNK = 128
EP = 102400
PAIR_PAD = 2 * EP
EB = 2048
KF = (BOND_DIM + 1) * ATOM_DIM

NHALF = 25600
NDUMMY = 128
ACC_ROWS = NHALF + NDUMMY


def _gather_call(src_pad, atom):
    per_w = EP // NW
    n_streams = per_w // CHUNK
    mesh = plsc.VectorSubcoreMesh(core_axis_name="c", subcore_axis_name="s")

    @functools.partial(
        pl.kernel,
        out_type=jax.ShapeDtypeStruct((EP, ATOM_DIM), jnp.float32),
        mesh=mesh,
        scratch_types=[
            pltpu.VMEM((per_w,), jnp.int32),
            pltpu.VMEM((per_w, ATOM_DIM), jnp.float32),
            pltpu.SemaphoreType.DMA,
        ],
        compiler_params=pltpu.CompilerParams(use_tc_tiling_on_sc=False, needs_layout_passes=False),
    )
    def gather_k(src_hbm, atom_hbm, nbr_hbm, idx_v, rows_v, sem):
        wid = lax.axis_index("s") * NC + lax.axis_index("c")
        ebase = wid * per_w
        pltpu.sync_copy(src_hbm.at[pl.ds(ebase, per_w)], idx_v)

        descs = [
            pltpu.async_copy(
                atom_hbm.at[idx_v.at[pl.ds(r * CHUNK, CHUNK)]],
                rows_v.at[pl.ds(r * CHUNK, CHUNK)],
                sem,
            )
            for r in range(n_streams)
        ]
        for d in descs:
            d.wait()
        pltpu.sync_copy(rows_v, nbr_hbm.at[pl.ds(ebase, per_w)])

    return gather_k(src_pad, atom)


def _dense_call(bond_t_all, nbr, w2t):
    last_block = (bond_t_all.shape[1] - 1) // EB

    def body(bond_ref, nbr_ref, w2t_ref, out_ref, ft_ref):
        nbr_t = nbr_ref[...].T
        bond_t = bond_ref[...]
        for c in range(BOND_DIM):
            ft_ref[c * ATOM_DIM:(c + 1) * ATOM_DIM, :] = (
                nbr_t * bond_t[c:c + 1, :]).astype(jnp.bfloat16)
        ft_ref[BOND_DIM * ATOM_DIM:, :] = nbr_t.astype(jnp.bfloat16)
        t_t = jnp.dot(w2t_ref[...], ft_ref[...],
                      preferred_element_type=jnp.float32)
        out_ref[...] = t_t.T

    return pl.pallas_call(
        body,
        grid=(EP // EB,),
        in_specs=[
            pl.BlockSpec((BOND_DIM, EB),
                         lambda i: (0, jnp.minimum(i, last_block))),
            pl.BlockSpec((EB, ATOM_DIM), lambda i: (i, 0)),
            pl.BlockSpec((ATOM_DIM, KF), lambda i: (0, 0)),
        ],
        out_specs=pl.BlockSpec((EB, ATOM_DIM), lambda i: (i, 0)),
        out_shape=jax.ShapeDtypeStruct((EP, ATOM_DIM), jnp.float32),
        scratch_shapes=[pltpu.VMEM((KF, EB), jnp.bfloat16)],
    )(bond_t_all, nbr, w2t)


def _scatter_call(dst_pad, transformed):
    per_t = EP // NS
    n_slabs = 5
    slab = per_t // n_slabs
    idx_rows = per_t // CHUNK
    zc0 = NHALF // NS
    zc1_total = N_NODES - NHALF
    zc1 = zc1_total // NS
    mesh = plsc.VectorSubcoreMesh(core_axis_name="c", subcore_axis_name="s")

    @functools.partial(
        pl.kernel,
        out_type=jax.ShapeDtypeStruct((N_NODES, ATOM_DIM), jnp.float32),
        mesh=mesh,
        scratch_types=[
            pltpu.VMEM_SHARED((ACC_ROWS, ATOM_DIM), jnp.float32),
            pltpu.VMEM((per_t,), jnp.int32),
            pltpu.VMEM((idx_rows, CHUNK), jnp.int32),
            pltpu.VMEM((slab, ATOM_DIM), jnp.float32),
            pltpu.SemaphoreType.DMA,
        ],
        compiler_params=pltpu.CompilerParams(use_tc_tiling_on_sc=False, needs_layout_passes=False),
    )
    def scatter_k(dst_hbm, t_hbm, out_hbm, acc, dst_v, idx_v, rows_v, sem):
        cid = lax.axis_index("c")
        sid = lax.axis_index("s")
        lanes = lax.iota(jnp.int32, 16)

        def zvec(i, _):
            rows_v[i // 2, pl.ds((i % 2) * 16, 16)] = jnp.zeros(
                (16,), jnp.float32)
            return 0

        lax.fori_loop(0, 1280, zvec, 0)
        zbase = sid * zc0
        for (off, n) in ((0, 640), (640, 640), (1280, 320)):
            pltpu.sync_copy(rows_v.at[pl.ds(0, n)],
                            acc.at[pl.ds(zbase + off, n)])

        plsc.subcore_barrier()

        pltpu.sync_copy(dst_hbm.at[pl.ds(sid * per_t, per_t)], dst_v)
        ebase = sid * per_t
        nlo = cid * NHALF

        def cbody(j, _):
            e = j * 16 + lanes
            d = dst_v[pl.ds(j * 16, 16)]
            local = d - nlo
            valid = ((local >= 0) & (local < NHALF)
                     & (ebase + e < N_EDGES))
            lv = jnp.where(valid, local, NHALF + (e & (NDUMMY - 1)))
            idx_v[j // 8, pl.ds((j % 8) * 16, 16)] = lv
            return 0

        lax.fori_loop(0, per_t // 16, cbody, 0)

        for sl in range(n_slabs):
            pltpu.sync_copy(t_hbm.at[pl.ds(ebase + sl * slab, slab)], rows_v)
            descs = [
                pltpu.async_copy(
                    rows_v.at[pl.ds(r * CHUNK, CHUNK)],
                    acc.at[idx_v.at[sl * (slab // CHUNK) + r]],
                    sem,
                    add=True,
                )
                for r in range(slab // CHUNK)
            ]
            for d in descs:
                d.wait()

        plsc.subcore_barrier()

        @pl.when(cid == 0)
        def _():
            for (off, n) in ((0, 640), (640, 640), (1280, 320)):
                r = sid * zc0 + off
                pltpu.sync_copy(acc.at[pl.ds(r, n)], rows_v.at[pl.ds(0, n)])
                pltpu.sync_copy(rows_v.at[pl.ds(0, n)],
                                out_hbm.at[pl.ds(r, n)])

        @pl.when(cid == 1)
        def _():
            for (off, n) in ((0, 640), (640, 640), (1280, 245)):
                r = sid * zc1 + off
                pltpu.sync_copy(acc.at[pl.ds(r, n)], rows_v.at[pl.ds(0, n)])
                pltpu.sync_copy(rows_v.at[pl.ds(0, n)],
                                out_hbm.at[pl.ds(NHALF + r, n)])

    return scatter_k(dst_pad, transformed)


def kernel(atom_features, bond_features, pair_indices, W, b):
    E = pair_indices.shape[0]
    zpad = jnp.zeros((EP - E,), jnp.int32)
    src_pad = jnp.concatenate([pair_indices[:, 1], zpad])
    dst_pad = jnp.concatenate([pair_indices[:, 0], zpad])

    w_aug = jnp.concatenate([W, b[:, None]], axis=1)
    w3 = w_aug.reshape(ATOM_DIM, ATOM_DIM, BOND_DIM + 1).transpose(2, 1, 0)
    w2t = w3.reshape(KF, ATOM_DIM).T.astype(jnp.bfloat16)

    nbr = _gather_call(src_pad, atom_features)
    transformed = _dense_call(bond_features.T, nbr, w2t)
    return _scatter_call(dst_pad, transformed)


# speedup vs baseline: 5.1839x; 1.0440x over previous
import functools

import jax
import jax.numpy as jnp
from jax import lax
from jax.experimental import pallas as pl
from jax.experimental.pallas import tpu as pltpu
from jax.experimental.pallas import tpu_sc as plsc

ATOM_DIM = 32
BOND_DIM = 16
N_NODES = 50000
N_EDGES = 100000

NC, NS = 2, 16
NW = NC * NS
CHUNK = 128
EP = 102400
PAIR_PAD = 2 * EP
EB = 4096
KF = (BOND_DIM + 1) * ATOM_DIM

NHALF = 25600
NDUMMY = 128
ACC_ROWS = NHALF + NDUMMY


def _gather_call(src_pad, atom):
    per_w = EP // NW
    n_streams = per_w // CHUNK
    mesh = plsc.VectorSubcoreMesh(core_axis_name="c", subcore_axis_name="s")

    @functools.partial(
        pl.kernel,
        out_type=jax.ShapeDtypeStruct((EP, ATOM_DIM), jnp.float32),
        mesh=mesh,
        scratch_types=[
            pltpu.VMEM((per_w,), jnp.int32),
            pltpu.VMEM((per_w, ATOM_DIM), jnp.float32),
            pltpu.SemaphoreType.DMA,
        ],
        compiler_params=pltpu.CompilerParams(use_tc_tiling_on_sc=False, needs_layout_passes=False),
    )
    def gather_k(src_hbm, atom_hbm, nbr_hbm, idx_v, rows_v, sem):
        wid = lax.axis_index("s") * NC + lax.axis_index("c")
        ebase = wid * per_w
        pltpu.sync_copy(src_hbm.at[pl.ds(ebase, per_w)], idx_v)

        descs = [
            pltpu.async_copy(
                atom_hbm.at[idx_v.at[pl.ds(r * CHUNK, CHUNK)]],
                rows_v.at[pl.ds(r * CHUNK, CHUNK)],
                sem,
            )
            for r in range(n_streams)
        ]
        for d in descs:
            d.wait()
        pltpu.sync_copy(rows_v, nbr_hbm.at[pl.ds(ebase, per_w)])

    return gather_k(src_pad, atom)


def _dense_call(bond_t_all, nbr, w2t):
    last_block = (bond_t_all.shape[1] - 1) // EB

    def body(bond_ref, nbr_ref, w2t_ref, out_ref, ft_ref):
        nbr_t = nbr_ref[...].T
        bond_t = bond_ref[...]
        for c in range(BOND_DIM):
            ft_ref[c * ATOM_DIM:(c + 1) * ATOM_DIM, :] = (
                nbr_t * bond_t[c:c + 1, :]).astype(jnp.bfloat16)
        ft_ref[BOND_DIM * ATOM_DIM:, :] = nbr_t.astype(jnp.bfloat16)
        t_t = jnp.dot(w2t_ref[...], ft_ref[...],
                      preferred_element_type=jnp.float32)
        out_ref[...] = t_t.T

    return pl.pallas_call(
        body,
        grid=(EP // EB,),
        in_specs=[
            pl.BlockSpec((BOND_DIM, EB),
                         lambda i: (0, jnp.minimum(i, last_block))),
            pl.BlockSpec((EB, ATOM_DIM), lambda i: (i, 0)),
            pl.BlockSpec((ATOM_DIM, KF), lambda i: (0, 0)),
        ],
        out_specs=pl.BlockSpec((EB, ATOM_DIM), lambda i: (i, 0)),
        out_shape=jax.ShapeDtypeStruct((EP, ATOM_DIM), jnp.float32),
        scratch_shapes=[pltpu.VMEM((KF, EB), jnp.bfloat16)],
    )(bond_t_all, nbr, w2t)


def _scatter_call(dst_pad, transformed):
    per_t = EP // NS
    n_slabs = 5
    slab = per_t // n_slabs
    idx_rows = per_t // CHUNK
    zc0 = NHALF // NS
    zc1_total = N_NODES - NHALF
    zc1 = zc1_total // NS
    mesh = plsc.VectorSubcoreMesh(core_axis_name="c", subcore_axis_name="s")

    @functools.partial(
        pl.kernel,
        out_type=jax.ShapeDtypeStruct((N_NODES, ATOM_DIM), jnp.float32),
        mesh=mesh,
        scratch_types=[
            pltpu.VMEM_SHARED((ACC_ROWS, ATOM_DIM), jnp.float32),
            pltpu.VMEM((per_t,), jnp.int32),
            pltpu.VMEM((idx_rows, CHUNK), jnp.int32),
            pltpu.VMEM((slab, ATOM_DIM), jnp.float32),
            pltpu.SemaphoreType.DMA,
        ],
        compiler_params=pltpu.CompilerParams(use_tc_tiling_on_sc=False, needs_layout_passes=False),
    )
    def scatter_k(dst_hbm, t_hbm, out_hbm, acc, dst_v, idx_v, rows_v, sem):
        cid = lax.axis_index("c")
        sid = lax.axis_index("s")
        lanes = lax.iota(jnp.int32, 16)

        def zvec(i, _):
            rows_v[i // 2, pl.ds((i % 2) * 16, 16)] = jnp.zeros(
                (16,), jnp.float32)
            return 0

        lax.fori_loop(0, 1280, zvec, 0)
        zbase = sid * zc0
        for (off, n) in ((0, 640), (640, 640), (1280, 320)):
            pltpu.sync_copy(rows_v.at[pl.ds(0, n)],
                            acc.at[pl.ds(zbase + off, n)])

        plsc.subcore_barrier()

        pltpu.sync_copy(dst_hbm.at[pl.ds(sid * per_t, per_t)], dst_v)
        ebase = sid * per_t
        nlo = cid * NHALF

        def cbody(j, _):
            e = j * 16 + lanes
            d = dst_v[pl.ds(j * 16, 16)]
            local = d - nlo
            valid = ((local >= 0) & (local < NHALF)
                     & (ebase + e < N_EDGES))
            lv = jnp.where(valid, local, NHALF + (e & (NDUMMY - 1)))
            idx_v[j // 8, pl.ds((j % 8) * 16, 16)] = lv
            return 0

        lax.fori_loop(0, per_t // 16, cbody, 0)

        for sl in range(n_slabs):
            pltpu.sync_copy(t_hbm.at[pl.ds(ebase + sl * slab, slab)], rows_v)
            descs = [
                pltpu.async_copy(
                    rows_v.at[pl.ds(r * CHUNK, CHUNK)],
                    acc.at[idx_v.at[sl * (slab // CHUNK) + r]],
                    sem,
                    add=True,
                )
                for r in range(slab // CHUNK)
            ]
            for d in descs:
                d.wait()

        plsc.subcore_barrier()

        @pl.when(cid == 0)
        def _():
            for (off, n) in ((0, 640), (640, 640), (1280, 320)):
                r = sid * zc0 + off
                pltpu.sync_copy(acc.at[pl.ds(r, n)], rows_v.at[pl.ds(0, n)])
                pltpu.sync_copy(rows_v.at[pl.ds(0, n)],
                                out_hbm.at[pl.ds(r, n)])

        @pl.when(cid == 1)
        def _():
            for (off, n) in ((0, 640), (640, 640), (1280, 245)):
                r = sid * zc1 + off
                pltpu.sync_copy(acc.at[pl.ds(r, n)], rows_v.at[pl.ds(0, n)])
                pltpu.sync_copy(rows_v.at[pl.ds(0, n)],
                                out_hbm.at[pl.ds(NHALF + r, n)])

    return scatter_k(dst_pad, transformed)


def kernel(atom_features, bond_features, pair_indices, W, b):
    E = pair_indices.shape[0]
    zpad = jnp.zeros((EP - E,), jnp.int32)
    src_pad = jnp.concatenate([pair_indices[:, 1], zpad])
    dst_pad = jnp.concatenate([pair_indices[:, 0], zpad])

    w_aug = jnp.concatenate([W, b[:, None]], axis=1)
    w3 = w_aug.reshape(ATOM_DIM, ATOM_DIM, BOND_DIM + 1).transpose(2, 1, 0)
    w2t = w3.reshape(KF, ATOM_DIM).T.astype(jnp.bfloat16)

    nbr = _gather_call(src_pad, atom_features)
    transformed = _dense_call(bond_features.T, nbr, w2t)
    return _scatter_call(dst_pad, transformed)


# speedup vs baseline: 5.1851x; 1.0002x over previous
import functools

import jax
import jax.numpy as jnp
from jax import lax
from jax.experimental import pallas as pl
from jax.experimental.pallas import tpu as pltpu
from jax.experimental.pallas import tpu_sc as plsc

ATOM_DIM = 32
BOND_DIM = 16
N_NODES = 50000
N_EDGES = 100000

NC, NS = 2, 16
NW = NC * NS
CHUNK = 128
EP = 102400
EB = 4096
KF = (BOND_DIM + 1) * ATOM_DIM

NHALF = 25600
NDUMMY = 128
ACC_ROWS = NHALF + NDUMMY


def _gather_call(src_pad, atom):
    per_w = EP // NW
    n_streams = per_w // CHUNK
    mesh = plsc.VectorSubcoreMesh(core_axis_name="c", subcore_axis_name="s")

    @functools.partial(
        pl.kernel,
        out_type=jax.ShapeDtypeStruct((EP, ATOM_DIM), jnp.float32),
        mesh=mesh,
        scratch_types=[
            pltpu.VMEM((per_w,), jnp.int32),
            pltpu.VMEM((per_w, ATOM_DIM), jnp.float32),
            pltpu.SemaphoreType.DMA,
        ],
        compiler_params=pltpu.CompilerParams(use_tc_tiling_on_sc=False, needs_layout_passes=False),
    )
    def gather_k(src_hbm, atom_hbm, nbr_hbm, idx_v, rows_v, sem):
        wid = lax.axis_index("s") * NC + lax.axis_index("c")
        ebase = wid * per_w
        pltpu.sync_copy(src_hbm.at[pl.ds(ebase, per_w)], idx_v)

        descs = [
            pltpu.async_copy(
                atom_hbm.at[idx_v.at[pl.ds(r * CHUNK, CHUNK)]],
                rows_v.at[pl.ds(r * CHUNK, CHUNK)],
                sem,
            )
            for r in range(n_streams)
        ]
        for d in descs:
            d.wait()
        pltpu.sync_copy(rows_v, nbr_hbm.at[pl.ds(ebase, per_w)])

    return gather_k(src_pad, atom)


def _dense_call(bond_t_all, nbr, w2t):
    last_block = (bond_t_all.shape[1] - 1) // EB

    def body(bond_ref, nbr_ref, w2t_ref, out_ref, ft_ref):
        nbr_t = nbr_ref[...].T
        bond_t = bond_ref[...]
        for c in range(BOND_DIM):
            ft_ref[c * ATOM_DIM:(c + 1) * ATOM_DIM, :] = (
                nbr_t * bond_t[c:c + 1, :]).astype(jnp.bfloat16)
        ft_ref[BOND_DIM * ATOM_DIM:, :] = nbr_t.astype(jnp.bfloat16)
        t_t = jnp.dot(w2t_ref[...], ft_ref[...],
                      preferred_element_type=jnp.float32)
        out_ref[...] = t_t.T

    return pl.pallas_call(
        body,
        grid=(EP // EB,),
        in_specs=[
            pl.BlockSpec((BOND_DIM, EB),
                         lambda i: (0, jnp.minimum(i, last_block))),
            pl.BlockSpec((EB, ATOM_DIM), lambda i: (i, 0)),
            pl.BlockSpec((ATOM_DIM, KF), lambda i: (0, 0)),
        ],
        out_specs=pl.BlockSpec((EB, ATOM_DIM), lambda i: (i, 0)),
        out_shape=jax.ShapeDtypeStruct((EP, ATOM_DIM), jnp.float32),
        scratch_shapes=[pltpu.VMEM((KF, EB), jnp.bfloat16)],
    )(bond_t_all, nbr, w2t)


def _scatter_call(dst_pad, transformed):
    per_t = EP // NS
    n_slabs = 5
    slab = per_t // n_slabs
    idx_rows = per_t // CHUNK
    zc0 = NHALF // NS
    zc1_total = N_NODES - NHALF
    zc1 = zc1_total // NS
    mesh = plsc.VectorSubcoreMesh(core_axis_name="c", subcore_axis_name="s")

    @functools.partial(
        pl.kernel,
        out_type=jax.ShapeDtypeStruct((N_NODES, ATOM_DIM), jnp.float32),
        mesh=mesh,
        scratch_types=[
            pltpu.VMEM_SHARED((ACC_ROWS, ATOM_DIM), jnp.float32),
            pltpu.VMEM((per_t,), jnp.int32),
            pltpu.VMEM((idx_rows, CHUNK), jnp.int32),
            pltpu.VMEM((slab, ATOM_DIM), jnp.float32),
            pltpu.SemaphoreType.DMA,
        ],
        compiler_params=pltpu.CompilerParams(use_tc_tiling_on_sc=False, needs_layout_passes=False),
    )
    def scatter_k(dst_hbm, t_hbm, out_hbm, acc, dst_v, idx_v, rows_v, sem):
        cid = lax.axis_index("c")
        sid = lax.axis_index("s")
        lanes = lax.iota(jnp.int32, 16)

        def zvec(i, _):
            rows_v[i // 2, pl.ds((i % 2) * 16, 16)] = jnp.zeros(
                (16,), jnp.float32)
            return 0

        lax.fori_loop(0, 1280, zvec, 0)
        zbase = sid * zc0
        for (off, n) in ((0, 640), (640, 640), (1280, 320)):
            pltpu.sync_copy(rows_v.at[pl.ds(0, n)],
                            acc.at[pl.ds(zbase + off, n)])

        plsc.subcore_barrier()

        pltpu.sync_copy(dst_hbm.at[pl.ds(sid * per_t, per_t)], dst_v)
        ebase = sid * per_t
        nlo = cid * NHALF

        def cbody(j, _):
            e = j * 16 + lanes
            d = dst_v[pl.ds(j * 16, 16)]
            local = d - nlo
            valid = ((local >= 0) & (local < NHALF)
                     & (ebase + e < N_EDGES))
            lv = jnp.where(valid, local, NHALF + (e & (NDUMMY - 1)))
            idx_v[j // 8, pl.ds((j % 8) * 16, 16)] = lv
            return 0

        lax.fori_loop(0, per_t // 16, cbody, 0)

        for sl in range(n_slabs):
            pltpu.sync_copy(t_hbm.at[pl.ds(ebase + sl * slab, slab)], rows_v)
            descs = [
                pltpu.async_copy(
                    rows_v.at[pl.ds(r * CHUNK, CHUNK)],
                    acc.at[idx_v.at[sl * (slab // CHUNK) + r]],
                    sem,
                    add=True,
                )
                for r in range(slab // CHUNK)
            ]
            for d in descs:
                d.wait()

        plsc.subcore_barrier()

        @pl.when(cid == 0)
        def _():
            for (off, n) in ((0, 640), (640, 640), (1280, 320)):
                r = sid * zc0 + off
                pltpu.sync_copy(acc.at[pl.ds(r, n)], rows_v.at[pl.ds(0, n)])
                pltpu.sync_copy(rows_v.at[pl.ds(0, n)],
                                out_hbm.at[pl.ds(r, n)])

        @pl.when(cid == 1)
        def _():
            for (off, n) in ((0, 640), (640, 640), (1280, 245)):
                r = sid * zc1 + off
                pltpu.sync_copy(acc.at[pl.ds(r, n)], rows_v.at[pl.ds(0, n)])
                pltpu.sync_copy(rows_v.at[pl.ds(0, n)],
                                out_hbm.at[pl.ds(NHALF + r, n)])

    return scatter_k(dst_pad, transformed)


def kernel(atom_features, bond_features, pair_indices, W, b):
    E = pair_indices.shape[0]
    zpad = jnp.zeros((EP - E,), jnp.int32)
    src_pad = jnp.concatenate([pair_indices[:, 1], zpad])
    dst_pad = jnp.concatenate([pair_indices[:, 0], zpad])

    w_aug = jnp.concatenate([W, b[:, None]], axis=1)
    w3 = w_aug.reshape(ATOM_DIM, ATOM_DIM, BOND_DIM + 1).transpose(2, 1, 0)
    w2t = w3.reshape(KF, ATOM_DIM).T.astype(jnp.bfloat16)

    nbr = _gather_call(src_pad, atom_features)
    transformed = _dense_call(bond_features.T, nbr, w2t)
    return _scatter_call(dst_pad, transformed)
